# baseline (speedup 1.0000x reference)
import functools

import jax
import jax.numpy as jnp
from jax import lax
from jax.experimental import pallas as pl
from jax.experimental.pallas import tpu as pltpu
from jax.experimental.pallas import tpu_sc as plsc

_BATCH = 4096
_VOCAB = 100000
_DIM = 64
_NC = 2
_NS = 16
_NW = _NC * _NS
_BPW = _BATCH // _NW
_LANES = 16


@functools.partial(
    pl.kernel,
    mesh=plsc.VectorSubcoreMesh(core_axis_name="c", subcore_axis_name="s"),
    out_type=jax.ShapeDtypeStruct((_BATCH * _DIM,), jnp.float32),
    scratch_types=[
        pltpu.VMEM((_BPW,), jnp.int32),
        pltpu.VMEM((_BPW, _DIM), jnp.float32),
        pltpu.VMEM((_BPW * _DIM,), jnp.float32),
        pltpu.SemaphoreType.DMA,
    ],
    compiler_params=pltpu.CompilerParams(use_tc_tiling_on_sc=False),
)
def _gather_add(x_hbm, table_hbm, idx_hbm, out_hbm, idx_v, rows_v, x_v, sem):
    wid = lax.axis_index("s") * _NC + lax.axis_index("c")
    base = wid * _BPW
    pltpu.sync_copy(idx_hbm.at[pl.ds(base, _BPW)], idx_v)
    gather = pltpu.async_copy(table_hbm.at[idx_v], rows_v, sem)
    pltpu.sync_copy(x_hbm.at[pl.ds(base * _DIM, _BPW * _DIM)], x_v)
    gather.wait()

    def body(i, carry):
        for j in range(_DIM // _LANES):
            sl = pl.ds(j * _LANES, _LANES)
            x_v[pl.ds(i * _DIM + j * _LANES, _LANES)] += rows_v[i, sl]
        return carry

    lax.fori_loop(0, _BPW, body, 0)
    pltpu.sync_copy(x_v, out_hbm.at[pl.ds(base * _DIM, _BPW * _DIM)])


def kernel(x, const, indices):
    out = _gather_add(x.reshape(-1), const, indices.astype(jnp.int32))
    return out.reshape(_BATCH, _DIM)


# speedup vs baseline: 2.4490x; 2.4490x over previous
import functools

import jax
import jax.numpy as jnp
from jax import lax
from jax.experimental import pallas as pl
from jax.experimental.pallas import tpu as pltpu
from jax.experimental.pallas import tpu_sc as plsc

_BATCH = 4096
_VOCAB = 100000
_DIM = 64
_NC = 2
_NS = 16
_NW = _NC * _NS
_RPW = _DIM // _NW
_LANES = 16


@functools.partial(
    pl.kernel,
    mesh=plsc.VectorSubcoreMesh(core_axis_name="c", subcore_axis_name="s"),
    out_type=jax.ShapeDtypeStruct((_DIM, _BATCH), jnp.float32),
    scratch_types=[
        pltpu.VMEM((_BATCH,), jnp.int32),
        pltpu.VMEM((_VOCAB,), jnp.float32),
        pltpu.VMEM((_BATCH,), jnp.float32),
        pltpu.VMEM((_BATCH,), jnp.float32),
    ],
    compiler_params=pltpu.CompilerParams(needs_layout_passes=False),
)
def _gather_add(xt_hbm, tablet_hbm, idx_hbm, outt_hbm, idx_v, row_v, x_v, o_v):
    wid = lax.axis_index("s") * _NC + lax.axis_index("c")
    pltpu.sync_copy(idx_hbm, idx_v)
    for t in range(_RPW):
        c = wid * _RPW + t
        pltpu.sync_copy(tablet_hbm.at[c], row_v)
        pltpu.sync_copy(xt_hbm.at[c], x_v)

        def body(g, carry):
            sl = pl.ds(g * _LANES, _LANES)
            vals = plsc.load_gather(row_v, [idx_v[sl]])
            o_v[sl] = x_v[sl] + vals
            return carry

        lax.fori_loop(0, _BATCH // _LANES, body, 0)
        pltpu.sync_copy(o_v, outt_hbm.at[c])


def kernel(x, const, indices):
    out_t = _gather_add(x.T, const.T, indices.astype(jnp.int32))
    return out_t.T


# speedup vs baseline: 2.5725x; 1.0504x over previous
import functools

import jax
import jax.numpy as jnp
from jax import lax
from jax.experimental import pallas as pl
from jax.experimental.pallas import tpu as pltpu
from jax.experimental.pallas import tpu_sc as plsc

_BATCH = 4096
_VOCAB = 100000
_DIM = 64
_NC = 2
_NS = 16
_NW = _NC * _NS
_RPW = _DIM // _NW
_LANES = 16
_H0 = 50048
_H1 = _VOCAB - _H0
_GROUPS = _BATCH // _LANES


@functools.partial(
    pl.kernel,
    mesh=plsc.VectorSubcoreMesh(core_axis_name="c", subcore_axis_name="s"),
    out_type=jax.ShapeDtypeStruct((_DIM, _BATCH), jnp.float32),
    scratch_types=[
        pltpu.VMEM((_BATCH,), jnp.int32),
        pltpu.VMEM((_H0,), jnp.float32),
        pltpu.VMEM((_H1,), jnp.float32),
        pltpu.VMEM((_BATCH,), jnp.float32),
        pltpu.VMEM((_BATCH,), jnp.float32),
        pltpu.VMEM((_BATCH,), jnp.float32),
        pltpu.SemaphoreType.DMA,
        pltpu.SemaphoreType.DMA,
    ],
    compiler_params=pltpu.CompilerParams(needs_layout_passes=False),
)
def _gather_add(xt_hbm, tablet_hbm, idx_hbm, outt_hbm,
                idx_v, buf0, buf1, tmp_v, x_v, o_v, semA, semB):
    wid = lax.axis_index("s") * _NC + lax.axis_index("c")
    c0 = wid * _RPW

    cpA = pltpu.async_copy(tablet_hbm.at[c0, pl.ds(0, _H0)], buf0, semA)
    cpB = pltpu.async_copy(tablet_hbm.at[c0, pl.ds(_H0, _H1)], buf1, semB)
    pltpu.sync_copy(idx_hbm, idx_v)

    def pass_low(g, carry):
        sl = pl.ds(g * _LANES, _LANES)
        i0 = jnp.minimum(idx_v[sl], _H0 - 1)
        tmp_v[sl] = plsc.load_gather(buf0, [i0])
        return carry

    def pass_high(g, carry):
        sl = pl.ds(g * _LANES, _LANES)
        iv = idx_v[sl]
        i1 = jnp.minimum(jnp.maximum(iv, _H0) - _H0, _H1 - 1)
        v1 = plsc.load_gather(buf1, [i1])
        o_v[sl] = x_v[sl] + jnp.where(iv < _H0, tmp_v[sl], v1)
        return carry

    for t in range(_RPW):
        c = c0 + t
        pltpu.sync_copy(xt_hbm.at[c], x_v)
        cpA.wait()
        lax.fori_loop(0, _GROUPS, pass_low, 0)
        if t + 1 < _RPW:
            cpA = pltpu.async_copy(
                tablet_hbm.at[c + 1, pl.ds(0, _H0)], buf0, semA)
        cpB.wait()
        lax.fori_loop(0, _GROUPS, pass_high, 0)
        if t + 1 < _RPW:
            cpB = pltpu.async_copy(
                tablet_hbm.at[c + 1, pl.ds(_H0, _H1)], buf1, semB)
        pltpu.sync_copy(o_v, outt_hbm.at[c])


def kernel(x, const, indices):
    out_t = _gather_add(x.T, const.T, indices.astype(jnp.int32))
    return out_t.T
